# Initial kernel scaffold; baseline (speedup 1.0000x reference)
import jax
import jax.numpy as jnp
from jax.experimental import pallas as pl


def kernel(x, edge_index, W1, b1, W2, b2, Wa, ba, Wb, bb):
    raise NotImplementedError("write your pallas kernel here")


# speedup vs baseline: 3.0121x; 3.0121x over previous
import functools

import jax
import jax.numpy as jnp
from jax import lax
from jax.experimental import pallas as pl
from jax.experimental.pallas import tpu as pltpu
from jax.experimental.pallas import tpu_sc as plsc

N = 10000
D = 128
E = 320000
HID = 128

NC = 2
NS = 16
NW = NC * NS
CH = 128
EP = 323584
RIDX = EP // CH
RW = RIDX // NW
NPAD = 10240
ZROWS = NPAD // NS

_mesh = plsc.VectorSubcoreMesh(core_axis_name="c", subcore_axis_name="s")


@functools.partial(
    pl.kernel,
    out_type=jax.ShapeDtypeStruct((NC, NPAD, D), jnp.float32),
    mesh=_mesh,
    scratch_types=[
        pltpu.VMEM((CH,), jnp.int32),
        pltpu.VMEM((CH,), jnp.int32),
        pltpu.VMEM((CH, D), jnp.float32),
        pltpu.VMEM_SHARED((NPAD, D), jnp.float32),
        pltpu.SemaphoreType.DMA,
    ],
)
def _segsum_sc(table_h, src_h, dst_h, out_h, sidx, didx, rows, acc, sem):
    c = lax.axis_index("c")
    s = lax.axis_index("s")
    wid = s * NC + c

    zv = jnp.zeros((16,), jnp.float32)

    def zbody(i, _):
        for j in range(D // 16):
            rows[i, pl.ds(j * 16, 16)] = zv
        return 0

    lax.fori_loop(0, CH, zbody, 0)
    for j in range(ZROWS // CH):
        pltpu.sync_copy(rows, acc.at[pl.ds(s * ZROWS + j * CH, CH)])
    plsc.subcore_barrier()

    def ebody(i, _):
        r = wid * RW + i
        pltpu.sync_copy(src_h.at[r], sidx)
        pltpu.sync_copy(dst_h.at[r], didx)
        pltpu.async_copy(table_h.at[sidx], rows, sem).wait()
        pltpu.sync_copy(rows, acc.at[didx], add=True)
        return 0

    lax.fori_loop(0, RW, ebody, 0)
    plsc.subcore_barrier()

    for j in range(ZROWS // CH):
        base = s * ZROWS + j * CH
        pltpu.sync_copy(acc.at[pl.ds(base, CH)], rows)
        pltpu.sync_copy(rows, out_h.at[c, pl.ds(base, CH)])


@functools.partial(
    pl.kernel,
    out_type=jax.ShapeDtypeStruct((EP, D), jnp.float32),
    mesh=_mesh,
    scratch_types=[
        pltpu.VMEM((CH,), jnp.int32),
        pltpu.VMEM((CH,), jnp.int32),
        pltpu.VMEM((CH, D), jnp.float32),
        pltpu.VMEM((CH, D), jnp.float32),
        pltpu.SemaphoreType.DMA,
        pltpu.SemaphoreType.DMA,
    ],
)
def _edge_emb_sc(table_h, src_h, dst_h, out_h, sidx, didx, ra, rb, sa, sb):
    c = lax.axis_index("c")
    s = lax.axis_index("s")
    wid = s * NC + c

    def ebody(i, _):
        r = wid * RW + i
        pltpu.sync_copy(src_h.at[r], sidx)
        pltpu.sync_copy(dst_h.at[r], didx)
        cpa = pltpu.async_copy(table_h.at[sidx], ra, sa)
        cpb = pltpu.async_copy(table_h.at[didx], rb, sb)
        cpa.wait()
        cpb.wait()

        def abody(k, _):
            for j in range(D // 16):
                sl = pl.ds(j * 16, 16)
                ra[k, sl] = ra[k, sl] + rb[k, sl]
            return 0

        lax.fori_loop(0, CH, abody, 0)
        pltpu.sync_copy(ra, out_h.at[pl.ds(r * CH, CH)])
        return 0

    lax.fori_loop(0, RW, ebody, 0)


def _lin1_body(x_ref, w_ref, o_ref):
    o_ref[...] = lax.dot_general(
        x_ref[...], w_ref[...], (((1,), (1,)), ((), ())),
        preferred_element_type=jnp.float32)


def _lin1(x, w):
    return pl.pallas_call(
        _lin1_body,
        grid=(5,),
        in_specs=[
            pl.BlockSpec((NPAD // 5, D), lambda i: (i, 0)),
            pl.BlockSpec((HID, D), lambda i: (0, 0)),
        ],
        out_specs=pl.BlockSpec((NPAD // 5, HID), lambda i: (i, 0)),
        out_shape=jax.ShapeDtypeStruct((NPAD, HID), jnp.float32),
    )(x, w)


def _conv_mid_body(xl_ref, p0_ref, p1_ref, b_ref, w_ref, o_ref):
    t = 0.5 * xl_ref[...] - (p0_ref[...] + p1_ref[...]) + b_ref[...]
    h1 = jnp.where(t > 0, t, jnp.exp(jnp.minimum(t, 0.0)) - 1.0)
    o_ref[...] = lax.dot_general(
        h1, w_ref[...], (((1,), (1,)), ((), ())),
        preferred_element_type=jnp.float32)


def _conv_mid(xl, parts, b1, w2):
    return pl.pallas_call(
        _conv_mid_body,
        grid=(5,),
        in_specs=[
            pl.BlockSpec((NPAD // 5, HID), lambda i: (i, 0)),
            pl.BlockSpec((NPAD // 5, HID), lambda i: (i, 0)),
            pl.BlockSpec((NPAD // 5, HID), lambda i: (i, 0)),
            pl.BlockSpec((1, HID), lambda i: (0, 0)),
            pl.BlockSpec((HID, HID), lambda i: (0, 0)),
        ],
        out_specs=pl.BlockSpec((NPAD // 5, HID), lambda i: (i, 0)),
        out_shape=jax.ShapeDtypeStruct((NPAD, HID), jnp.float32),
    )(xl, parts[0], parts[1], b1.reshape(1, HID), w2)


def _conv_out_body(hl_ref, p0_ref, p1_ref, b_ref, o_ref):
    o_ref[...] = 0.5 * hl_ref[...] - (p0_ref[...] + p1_ref[...]) + b_ref[...]


def _conv_out(hl, parts, b2):
    return pl.pallas_call(
        _conv_out_body,
        grid=(5,),
        in_specs=[
            pl.BlockSpec((NPAD // 5, HID), lambda i: (i, 0)),
            pl.BlockSpec((NPAD // 5, HID), lambda i: (i, 0)),
            pl.BlockSpec((NPAD // 5, HID), lambda i: (i, 0)),
            pl.BlockSpec((1, HID), lambda i: (0, 0)),
        ],
        out_specs=pl.BlockSpec((NPAD // 5, HID), lambda i: (i, 0)),
        out_shape=jax.ShapeDtypeStruct((NPAD, HID), jnp.float32),
    )(hl, parts[0], parts[1], b2.reshape(1, HID))


def _mlp_body(ee_ref, wa_ref, ba_ref, wb_ref, bb_ref, o_ref):
    h = lax.dot_general(
        ee_ref[...], wa_ref[...], (((1,), (1,)), ((), ())),
        preferred_element_type=jnp.float32)
    h = jnp.maximum(h + ba_ref[...], 0.0)
    logit = jnp.sum(h * wb_ref[...], axis=1, keepdims=True)
    o_ref[...] = jax.nn.sigmoid(logit + bb_ref[...])


def _edge_mlp(ee, wa, ba, wb, bb):
    blk = 2048
    return pl.pallas_call(
        _mlp_body,
        grid=(EP // blk,),
        in_specs=[
            pl.BlockSpec((blk, HID), lambda i: (i, 0)),
            pl.BlockSpec((2 * HID, HID), lambda i: (0, 0)),
            pl.BlockSpec((1, 2 * HID), lambda i: (0, 0)),
            pl.BlockSpec((1, 2 * HID), lambda i: (0, 0)),
            pl.BlockSpec((1, 1), lambda i: (0, 0)),
        ],
        out_specs=pl.BlockSpec((blk, 1), lambda i: (i, 0)),
        out_shape=jax.ShapeDtypeStruct((EP, 1), jnp.float32),
    )(ee, wa, ba.reshape(1, 2 * HID), wb.reshape(1, 2 * HID),
      bb.reshape(1, 1))


def kernel(x, edge_index, W1, b1, W2, b2, Wa, ba, Wb, bb):
    src = edge_index[0]
    dst = edge_index[1]
    pad = EP - E
    srcp = jnp.concatenate([src, jnp.zeros((pad,), jnp.int32)]).reshape(RIDX, CH)
    dstp = jnp.concatenate([dst, jnp.full((pad,), N, jnp.int32)]).reshape(RIDX, CH)

    xp = jnp.pad(x, ((0, NPAD - N), (0, 0)))
    xl = _lin1(xp, W1)
    parts1 = _segsum_sc(xl, srcp, dstp)
    hl = _conv_mid(xl, parts1, b1, W2)
    parts2 = _segsum_sc(hl, srcp, dstp)
    h = _conv_out(hl, parts2, b2)
    ee = _edge_emb_sc(h, srcp, dstp)
    ew = _edge_mlp(ee, Wa, ba, Wb, bb)
    return ew[:E, 0]
